# Initial kernel scaffold; baseline (speedup 1.0000x reference)
import jax
import jax.numpy as jnp
from jax.experimental import pallas as pl


def kernel(x, batch, smask, W, b):
    raise NotImplementedError("write your pallas kernel here")


# speedup vs baseline: 2.0954x; 2.0954x over previous
import jax
import jax.numpy as jnp
from jax import lax
from jax.experimental import pallas as pl
from jax.experimental.pallas import tpu as pltpu
from jax.experimental.pallas import tpu_sc as plsc

N = 100000
D = 128
G = 512
NC = 2
NS = 16
NW = NC * NS
BLK = 200
NBLOCKS = N // BLK
MAXB = (NBLOCKS + NW - 1) // NW
PB = 256


def _sc_body(x_hbm, bat_hbm, sm_hbm, w_hbm, bv_hbm, iden_hbm,
             wout_hbm, part_hbm,
             xbuf, bbuf, sbuf, wbuf, accloc, idenbuf, Wbuf, bvbuf, shacc):
    cid = lax.axis_index("c")
    sid = lax.axis_index("s")
    wid = sid * NC + cid

    zv = jnp.zeros((16,), jnp.float32)
    zi = jnp.zeros((16,), jnp.int32)
    lane = lax.iota(jnp.int32, 16)
    lane0 = lane == 0

    def _zrow(i, _):
        for j in range(8):
            accloc[i, pl.ds(16 * j, 16)] = zv
        return 0
    lax.fori_loop(0, G, _zrow, 0)

    pltpu.sync_copy(accloc.at[pl.ds(0, 32)], shacc.at[pl.ds(sid * 32, 32)])

    pltpu.sync_copy(w_hbm, Wbuf)
    pltpu.sync_copy(bv_hbm, bvbuf)
    pltpu.sync_copy(iden_hbm, idenbuf)
    Wv = [Wbuf[j, :] for j in range(8)]
    bv = bvbuf[:]

    plsc.subcore_barrier()

    def flush(seg, acc):
        for j in range(8):
            plsc.addupdate(accloc.at[seg, pl.ds(16 * j, 16)], acc[j])

    def do_block(k, _):
        bid = wid + NW * k

        @pl.when(bid < NBLOCKS)
        def _():
            row0 = bid * BLK
            pltpu.sync_copy(x_hbm.at[pl.ds(row0, BLK)], xbuf)
            pltpu.sync_copy(bat_hbm.at[pl.ds(row0, BLK)], bbuf.at[pl.ds(0, BLK)])
            pltpu.sync_copy(sm_hbm.at[pl.ds(row0, BLK)], sbuf.at[pl.ds(0, BLK)])

            def row(r, carry):
                cur_seg, acc = carry
                rv = jnp.broadcast_to(r, (16,))
                seg_vec = plsc.load_gather(bbuf, [rv])
                sm_vec = plsc.load_gather(sbuf, [rv])
                seg = seg_vec[0]

                xv = [xbuf[r, pl.ds(16 * j, 16)] for j in range(8)]
                p = xv[0] * Wv[0]
                for j in range(1, 8):
                    p = p + xv[j] * Wv[j]
                dot = jnp.sum(p)
                z = jnp.broadcast_to(dot, (16,)) + bv
                e = jnp.exp(-z)
                w_vec = sm_vec / (1.0 + e)

                plsc.store_scatter(wbuf, [rv], w_vec, mask=lane0)

                changed = seg != cur_seg

                @pl.when(changed)
                def _():
                    flush(cur_seg, acc)

                cvec = jnp.broadcast_to(changed, (16,))
                acc = [jnp.where(cvec, zv, acc[j]) + xv[j] * w_vec
                       for j in range(8)]
                return seg, acc

            seg0 = plsc.load_gather(bbuf, [zi])[0]
            fseg, facc = lax.fori_loop(0, BLK, row, (seg0, [zv] * 8))
            flush(fseg, facc)
            pltpu.sync_copy(wbuf.at[pl.ds(0, BLK)],
                            wout_hbm.at[pl.ds(row0, BLK)])
        return 0

    lax.fori_loop(0, MAXB, do_block, 0)

    pltpu.sync_copy(accloc, shacc.at[idenbuf], add=True)
    plsc.subcore_barrier()
    pltpu.sync_copy(shacc.at[pl.ds(sid * 32, 32)],
                    part_hbm.at[pl.ds(cid * G + sid * 32, 32)])


def _combine(parts_ref, o_ref):
    o_ref[...] = parts_ref[0:G, :] + parts_ref[G:2 * G, :]


@jax.jit
def kernel(x, batch, smask, W, b):
    bat2 = batch.astype(jnp.int32)
    Wf = W.reshape(8, 16)
    bvec = jnp.broadcast_to(b.astype(jnp.float32), (16,))
    iden = jnp.arange(G, dtype=jnp.int32)

    mesh = plsc.VectorSubcoreMesh(core_axis_name="c", subcore_axis_name="s",
                                  num_cores=NC, num_subcores=NS)
    sc = pl.kernel(
        _sc_body,
        out_type=(
            jax.ShapeDtypeStruct((N,), jnp.float32),
            jax.ShapeDtypeStruct((NC * G, D), jnp.float32),
        ),
        mesh=mesh,
        compiler_params=pltpu.CompilerParams(needs_layout_passes=False),
        scratch_types=[
            pltpu.VMEM((BLK, D), jnp.float32),
            pltpu.VMEM((PB,), jnp.int32),
            pltpu.VMEM((PB,), jnp.float32),
            pltpu.VMEM((PB,), jnp.float32),
            pltpu.VMEM((G, D), jnp.float32),
            pltpu.VMEM((G,), jnp.int32),
            pltpu.VMEM((8, 16), jnp.float32),
            pltpu.VMEM((16,), jnp.float32),
            pltpu.VMEM_SHARED((G, D), jnp.float32),
        ],
    )
    wout, parts = sc(x, bat2, smask, Wf, bvec, iden)

    h = pl.pallas_call(
        _combine,
        out_shape=jax.ShapeDtypeStruct((G, D), jnp.float32),
    )(parts)

    return h, wout.reshape(N, 1)
